# baseline (speedup 1.0000x reference)
import jax
import jax.numpy as jnp
from jax import lax
from jax.experimental import pallas as pl
from jax.experimental.pallas import tpu as pltpu
from jax.experimental.pallas import tpu_sc as plsc

NC = 2
NS = 16
NW = NC * NS

MAXLEN = 200
EMBED = 64
BATCH = 4096
SEQ = 200

BB = 128
NBT = BATCH // BB
NCHUNK = SEQ * NBT
CPT = NCHUNK // NW
NSLOT = 4


def _body(xt_hbm, tok_hbm, pos_hbm, out_hbm, idx_v, rows_v, w_v, pos_v,
          si, sg, so, ps):
    wid = lax.axis_index("s") * NC + lax.axis_index("c")
    c0 = wid * CPT

    pltpu.async_copy(pos_hbm, pos_v, ps).wait()

    ar = jnp.arange(16, dtype=jnp.int32)
    E8 = {e0: (ar + e0) // 8 for e0 in range(0, EMBED, 16)}
    ER = {e0: (ar + e0) % 8 for e0 in range(0, EMBED, 16)}

    def sb(k):
        c = c0 + k
        return c // NBT, c % NBT

    def issue_idx(k):
        s, bt = sb(k)
        pltpu.async_copy(xt_hbm.at[s, pl.ds(bt * BB, BB)], idx_v.at[k % NSLOT], si)

    def issue_gather(k):
        b = k % NSLOT
        pltpu.async_copy(tok_hbm.at[idx_v.at[b]], rows_v.at[b], sg)

    def wait_idx():
        pltpu.make_async_copy(xt_hbm.at[0, pl.ds(0, BB)], idx_v.at[0], si).wait()

    def wait_gather():
        pltpu.make_async_copy(tok_hbm.at[pl.ds(0, BB)], rows_v.at[0], sg).wait()

    def wait_out():
        pltpu.make_async_copy(out_hbm.at[0, 0, 0], w_v.at[0, 0], so).wait()

    issue_idx(0)
    wait_idx()
    issue_gather(0)
    issue_idx(1)

    def step(k, _):
        b = k % NSLOT
        s, bt = sb(k)

        @pl.when(k >= NSLOT)
        def _():
            for e8 in range(8):
                wait_out()

        @pl.when(k + 1 < CPT)
        def _():
            wait_idx()
            issue_gather(k + 1)

        wait_gather()

        def per_tok(t, _):
            tv = jnp.full((16,), t, jnp.int32)
            for e0 in range(0, EMBED, 16):
                vec = rows_v[b, t, pl.ds(e0, 16)] + pos_v[s, pl.ds(e0, 16)]
                plsc.store_scatter(w_v.at[b], [E8[e0], ER[e0], tv], vec)
            return 0

        lax.fori_loop(0, BB, per_tok, 0, unroll=4)

        for e8 in range(8):
            pltpu.async_copy(w_v.at[b, e8], out_hbm.at[s, e8, bt], so)

        @pl.when(k + 2 < CPT)
        def _():
            issue_idx(k + 2)

        return 0

    lax.fori_loop(0, CPT, step, 0)

    for _ in range(NSLOT * 8):
        wait_out()


@jax.jit
def _run(xt, token_table, pos_table):
    mesh = plsc.VectorSubcoreMesh(core_axis_name="c", subcore_axis_name="s")
    f = pl.kernel(
        _body,
        out_type=jax.ShapeDtypeStruct((SEQ, EMBED // 8, NBT, 8, BB), jnp.float32),
        mesh=mesh,
        scratch_types=[
            pltpu.VMEM((NSLOT, BB), jnp.int32),
            pltpu.VMEM((NSLOT, BB, EMBED), jnp.float32),
            pltpu.VMEM((NSLOT, EMBED // 8, 8, BB), jnp.float32),
            pltpu.VMEM((SEQ, EMBED), jnp.float32),
            pltpu.SemaphoreType.DMA,
            pltpu.SemaphoreType.DMA,
            pltpu.SemaphoreType.DMA,
            pltpu.SemaphoreType.DMA,
        ],
        compiler_params=pltpu.CompilerParams(use_tc_tiling_on_sc=False, needs_layout_passes=False),
    )
    return f(xt, token_table, pos_table)


def kernel(x, token_table, pos_table):
    xt = x.astype(jnp.int32).T
    out5 = _run(xt, token_table, pos_table)
    return out5.transpose(2, 4, 0, 1, 3).reshape(BATCH, SEQ, EMBED)
